# baseline (speedup 1.0000x reference)
import functools

import jax
import jax.numpy as jnp
from jax import lax
from jax.experimental import pallas as pl
from jax.experimental.pallas import tpu as pltpu
from jax.experimental.pallas import tpu_sc as plsc

_N_REL = 1000
_DIM = 32
_BATCH = 16384
_GAMMA = 12.0
_L = 16
_NC = 2
_NS = 16
_NW = _NC * _NS
_BPW = _BATCH // _NW
_NCHUNK = _BPW // _L
_IDX_CHUNK = 128
_TINY = 1e-24


def _rsqrt(s):
    i = plsc.bitcast(s, jnp.int32)
    i = jnp.int32(0x5F3759DF) - lax.shift_right_arithmetic(i, 1)
    y = plsc.bitcast(i, jnp.float32)
    for _ in range(4):
        y = y * (1.5 - 0.5 * s * y * y)
    return y


def _transd_body(h_hbm, r_hbm, t_hbm, ent_hbm, rel_hbm, pent_hbm, prel_hbm,
                 out_hbm,
                 h_v, r_v, t_v, rm_v,
                 hv_rows, tv_rows, hp_rows, tp_rows, rv_rows, rp_rows,
                 out_v, sem):
    wid = lax.axis_index("s") * _NC + lax.axis_index("c")
    base = wid * _BPW
    iota = lax.broadcasted_iota(jnp.int32, (_L,), 0)

    pltpu.sync_copy(h_hbm.at[pl.ds(base, _BPW)], h_v)
    pltpu.sync_copy(t_hbm.at[pl.ds(base, _BPW)], t_v)
    pltpu.sync_copy(r_hbm.at[pl.ds(base, _BPW)], r_v)

    copies = []
    for j in range(_BPW // _IDX_CHUNK):
        ids = pl.ds(j * _IDX_CHUNK, _IDX_CHUNK)
        rds = pl.ds(j * _IDX_CHUNK, _IDX_CHUNK)
        copies.append(pltpu.async_copy(ent_hbm.at[h_v.at[ids]], hv_rows.at[rds], sem))
        copies.append(pltpu.async_copy(ent_hbm.at[t_v.at[ids]], tv_rows.at[rds], sem))
        copies.append(pltpu.async_copy(pent_hbm.at[h_v.at[ids]], hp_rows.at[rds], sem))
        copies.append(pltpu.async_copy(pent_hbm.at[t_v.at[ids]], tp_rows.at[rds], sem))

    def _mod_chunk(c, carry):
        idx = c * _L + iota
        rr = plsc.load_gather(r_v, [idx])
        plsc.store_scatter(rm_v, [idx], lax.rem(rr, jnp.int32(_N_REL)))
        return carry

    lax.fori_loop(0, _NCHUNK, _mod_chunk, 0)

    for j in range(_BPW // _IDX_CHUNK):
        ids = pl.ds(j * _IDX_CHUNK, _IDX_CHUNK)
        rds = pl.ds(j * _IDX_CHUNK, _IDX_CHUNK)
        copies.append(pltpu.async_copy(rel_hbm.at[rm_v.at[ids]], rv_rows.at[rds], sem))
        copies.append(pltpu.async_copy(prel_hbm.at[rm_v.at[ids]], rp_rows.at[rds], sem))

    for cp in copies:
        cp.wait()

    def _chunk(c, carry):
        row = c * _L + iota
        zero = jnp.zeros((_L,), jnp.float32)
        shh = stt = srr = spp = sht = shr = shp = str_ = stp = srp = dh = dt = zero
        for d in range(_DIM):
            col = jnp.full((_L,), d, jnp.int32)
            hd = plsc.load_gather(hv_rows, [row, col])
            td = plsc.load_gather(tv_rows, [row, col])
            rd = plsc.load_gather(rv_rows, [row, col])
            pd = plsc.load_gather(rp_rows, [row, col])
            hpd = plsc.load_gather(hp_rows, [row, col])
            tpd = plsc.load_gather(tp_rows, [row, col])
            shh += hd * hd
            stt += td * td
            srr += rd * rd
            spp += pd * pd
            sht += hd * td
            shr += hd * rd
            shp += hd * pd
            str_ += td * rd
            stp += td * pd
            srp += rd * pd
            dh += hd * hpd
            dt += td * tpd

        a = _rsqrt(jnp.maximum(shh, _TINY))
        cc = _rsqrt(jnp.maximum(stt, _TINY))
        rin = _rsqrt(jnp.maximum(srr, _TINY))
        bh = a * dh
        bt = cc * dt
        yh = a * a * shh + 2.0 * a * bh * shp + bh * bh * spp
        yt = cc * cc * stt + 2.0 * cc * bt * stp + bt * bt * spp
        iyh = _rsqrt(jnp.maximum(yh, _TINY))
        iyt = _rsqrt(jnp.maximum(yt, _TINY))
        rr = plsc.load_gather(r_v, [row])
        sgn = jnp.where(rr < _N_REL, jnp.float32(1.0), jnp.float32(-1.0))
        ch = iyh * a
        ct = -(iyt * cc)
        cr = sgn * rin
        cp_ = iyh * bh - iyt * bt
        s = (ch * ch * shh + ct * ct * stt + cr * cr * srr + cp_ * cp_ * spp
             + 2.0 * (ch * ct * sht + ch * cr * shr + ch * cp_ * shp
                      + ct * cr * str_ + ct * cp_ * stp + cr * cp_ * srp))
        s = jnp.maximum(s, 0.0)
        dist = s * _rsqrt(jnp.maximum(s, _TINY))
        plsc.store_scatter(out_v, [row], _GAMMA - dist)
        return carry

    lax.fori_loop(0, _NCHUNK, _chunk, 0)

    pltpu.sync_copy(out_v, out_hbm.at[pl.ds(base, _BPW)])


_transd = pl.kernel(
    _transd_body,
    out_type=jax.ShapeDtypeStruct((_BATCH,), jnp.float32),
    mesh=plsc.VectorSubcoreMesh(core_axis_name="c", subcore_axis_name="s"),
    compiler_params=pltpu.CompilerParams(
        needs_layout_passes=False, use_tc_tiling_on_sc=False),
    scratch_types=[
        pltpu.VMEM((_BPW,), jnp.int32),
        pltpu.VMEM((_BPW,), jnp.int32),
        pltpu.VMEM((_BPW,), jnp.int32),
        pltpu.VMEM((_BPW,), jnp.int32),
        pltpu.VMEM((_BPW, _DIM), jnp.float32),
        pltpu.VMEM((_BPW, _DIM), jnp.float32),
        pltpu.VMEM((_BPW, _DIM), jnp.float32),
        pltpu.VMEM((_BPW, _DIM), jnp.float32),
        pltpu.VMEM((_BPW, _DIM), jnp.float32),
        pltpu.VMEM((_BPW, _DIM), jnp.float32),
        pltpu.VMEM((_BPW,), jnp.float32),
        pltpu.SemaphoreType.DMA,
    ],
)


def kernel(h, r, t, ent_embed, rel_embed, proj_ent_embed, proj_rel_embed):
    h = jnp.asarray(h, jnp.int32)
    r = jnp.asarray(r, jnp.int32)
    t = jnp.asarray(t, jnp.int32)
    return _transd(h, r, t, ent_embed, rel_embed, proj_ent_embed, proj_rel_embed)
